# baseline (speedup 1.0000x reference)
import functools

import jax
import jax.numpy as jnp
from jax import lax
from jax.experimental import pallas as pl
from jax.experimental.pallas import tpu as pltpu
from jax.experimental.pallas import tpu_sc as plsc

B, N1, N2, D, NS = 2, 2048, 2048, 64, 16
C1, C2 = 128, 128
BLK = 256
BLK2 = 128
NH = 2
N1H = N1 // NH
TOTH = B * N1H * NS
NW = 32
PER_W = TOTH // NW
CH = 128
NCH = PER_W // CH

_HI = jax.lax.Precision.HIGHEST


def _z_kernel(p2_ref, x2_ref, w1b_ref, w1c_ref, z_ref):
    x2 = x2_ref[0]
    w1c = w1c_ref[...]
    zc = (x2[:, 0:1] * w1c[0:1, :] + x2[:, 1:2] * w1c[1:2, :]
          + x2[:, 2:3] * w1c[2:3, :])
    z_ref[0] = jnp.dot(p2_ref[0], w1b_ref[...],
                       preferred_element_type=jnp.float32, precision=_HI) + zc


def _topk_kernel(x1_ref, x2_ref, idx_ref, w_ref):
    b = pl.program_id(0)
    x1 = x1_ref[0]
    x2 = x2_ref[0]
    x1sq = jnp.sum(x1 * x1, axis=1, keepdims=True)
    x2sq = jnp.sum(x2 * x2, axis=0, keepdims=True)
    x1b = x1.astype(jnp.bfloat16).astype(jnp.float32)
    x2b = x2.astype(jnp.bfloat16).astype(jnp.float32)
    cross_sel = (x1b[:, 0:1] * x2b[0:1, :] + x1b[:, 1:2] * x2b[1:2, :]
                 + x1b[:, 2:3] * x2b[2:3, :])
    sqd = (x1sq + x2sq) - 2.0 * cross_sel
    cross_ex = (x1[:, 0:1] * x2[0:1, :] + x1[:, 1:2] * x2[1:2, :]
                + x1[:, 2:3] * x2[2:3, :])
    sqde = (x1sq + x2sq) - 2.0 * cross_ex

    lane = jax.lax.broadcasted_iota(jnp.int32, (BLK, N2), 1)
    idx_cols = []
    w_cols = []
    for _ in range(NS):
        idx = jnp.argmin(sqd, axis=1).astype(jnp.int32)[:, None]
        hit = lane == idx
        sqd = jnp.where(hit, jnp.inf, sqd)
        me = jnp.sum(jnp.where(hit, sqde, 0.0), axis=1, keepdims=True)
        w = 1.0 / jnp.maximum(jnp.sqrt(jnp.maximum(me, 0.0)), 1e-10)
        idx_cols.append(idx + b * N2)
        w_cols.append(w)
    ws = jnp.concatenate(w_cols, axis=1)
    idx_ref[0] = jnp.concatenate(idx_cols, axis=1)
    w_ref[0] = ws / jnp.sum(ws, axis=1, keepdims=True)


def _mlp_kernel(gz_ref, x1_ref, p1_ref, w1a_ref, w1c_ref, b1_ref,
                w2_ref, b2_ref, wn_ref, out_ref):
    x1 = x1_ref[0]
    p1 = p1_ref[0]
    w1c = w1c_ref[...]
    basec = (x1[:, 0:1] * w1c[0:1, :] + x1[:, 1:2] * w1c[1:2, :]
             + x1[:, 2:3] * w1c[2:3, :])
    base = (jnp.dot(p1, w1a_ref[...], preferred_element_type=jnp.float32,
                    precision=_HI) - basec + b1_ref[...])
    base_e = jnp.broadcast_to(base[:, None, :], (BLK2, NS, C1)).reshape(
        BLK2 * NS, C1)
    h1 = gz_ref[0] + base_e
    h1 = jnp.where(h1 >= 0, h1, 0.1 * h1)
    h2 = jnp.dot(h1, w2_ref[...], preferred_element_type=jnp.float32,
                 precision=_HI) + b2_ref[...]
    h2 = jnp.where(h2 >= 0, h2, 0.1 * h2)
    h2 = h2.reshape(BLK2, NS, C2)
    out_ref[0] = jnp.sum(wn_ref[0][:, :, None] * h2, axis=1)


def _sc_gather(table, idx3):
    mesh = plsc.VectorSubcoreMesh(core_axis_name="c", subcore_axis_name="s")

    @functools.partial(
        pl.kernel, mesh=mesh,
        out_type=jax.ShapeDtypeStruct((TOTH, C1), jnp.float32),
        scratch_types=[
            pltpu.VMEM((NCH, CH), jnp.int32),
            pltpu.VMEM((CH, C1), jnp.float32),
            pltpu.VMEM((CH, C1), jnp.float32),
            pltpu.SemaphoreType.DMA,
            pltpu.SemaphoreType.DMA,
        ],
    )
    def k(table_hbm, idx_hbm, out_hbm, idx_v, rows_a, rows_b, sem_a, sem_b):
        wid = lax.axis_index("s") * 2 + lax.axis_index("c")
        base = wid * PER_W
        pltpu.sync_copy(idx_hbm.at[wid], idx_v)
        bufs = (rows_a, rows_b)
        sems = (sem_a, sem_b)
        cps = [None, None]
        for c in range(NCH + 1):
            if c < NCH:
                cps[c % 2] = pltpu.async_copy(table_hbm.at[idx_v.at[c]],
                                              bufs[c % 2], sems[c % 2])
            if c >= 1:
                cps[(c - 1) % 2].wait()
                pltpu.sync_copy(bufs[(c - 1) % 2],
                                out_hbm.at[pl.ds(base + (c - 1) * CH, CH)])

    return k(table, idx3)


def _half(z2d, xyz2, x1t_h, p1t_h, w1a, w1c, b1r, w2t, b2r):
    knn_idx, wn = pl.pallas_call(
        _topk_kernel,
        grid=(B, N1H // BLK),
        in_specs=[
            pl.BlockSpec((1, BLK, 3), lambda b, i: (b, i, 0)),
            pl.BlockSpec((1, 3, N2), lambda b, i: (b, 0, 0)),
        ],
        out_specs=(
            pl.BlockSpec((1, BLK, NS), lambda b, i: (b, i, 0)),
            pl.BlockSpec((1, BLK, NS), lambda b, i: (b, i, 0)),
        ),
        out_shape=(
            jax.ShapeDtypeStruct((B, N1H, NS), jnp.int32),
            jax.ShapeDtypeStruct((B, N1H, NS), jnp.float32),
        ),
    )(x1t_h, xyz2)

    gz = _sc_gather(z2d, knn_idx.reshape(NW, NCH, CH))

    return pl.pallas_call(
        _mlp_kernel,
        grid=(B, N1H // BLK2),
        in_specs=[
            pl.BlockSpec((1, BLK2 * NS, C1), lambda b, i: (b, i, 0)),
            pl.BlockSpec((1, BLK2, 3), lambda b, i: (b, i, 0)),
            pl.BlockSpec((1, BLK2, D), lambda b, i: (b, i, 0)),
            pl.BlockSpec((D, C1), lambda b, i: (0, 0)),
            pl.BlockSpec((3, C1), lambda b, i: (0, 0)),
            pl.BlockSpec((1, C1), lambda b, i: (0, 0)),
            pl.BlockSpec((C1, C2), lambda b, i: (0, 0)),
            pl.BlockSpec((1, C2), lambda b, i: (0, 0)),
            pl.BlockSpec((1, BLK2, NS), lambda b, i: (b, i, 0)),
        ],
        out_specs=pl.BlockSpec((1, BLK2, C2), lambda b, i: (b, i, 0)),
        out_shape=jax.ShapeDtypeStruct((B, N1H, C2), jnp.float32),
    )(gz.reshape(B, N1H * NS, C1), x1t_h, p1t_h, w1a, w1c, b1r, w2t, b2r, wn)


@jax.jit
def kernel(xyz1, xyz2, points1, points2, W1, b1, W2, b2):
    x1t = jnp.transpose(xyz1, (0, 2, 1))
    x2t = jnp.transpose(xyz2, (0, 2, 1))
    p1t = jnp.transpose(points1, (0, 2, 1))
    p2t = jnp.transpose(points2, (0, 2, 1))
    w1a = W1[:, :D].T
    w1b = W1[:, D:2 * D].T
    w1c = W1[:, 2 * D:].T
    w2t = W2.T
    b1r = b1.reshape(1, C1)
    b2r = b2.reshape(1, C2)

    z = pl.pallas_call(
        _z_kernel,
        grid=(B,),
        in_specs=[
            pl.BlockSpec((1, N2, D), lambda b: (b, 0, 0)),
            pl.BlockSpec((1, N2, 3), lambda b: (b, 0, 0)),
            pl.BlockSpec((D, C1), lambda b: (0, 0)),
            pl.BlockSpec((3, C1), lambda b: (0, 0)),
        ],
        out_specs=pl.BlockSpec((1, N2, C1), lambda b: (b, 0, 0)),
        out_shape=jax.ShapeDtypeStruct((B, N2, C1), jnp.float32),
    )(p2t, x2t, w1b, w1c)
    z2d = z.reshape(B * N2, C1)

    outs = [
        _half(z2d, xyz2, x1t[:, h * N1H:(h + 1) * N1H],
              p1t[:, h * N1H:(h + 1) * N1H], w1a, w1c, b1r, w2t, b2r)
        for h in range(NH)
    ]
    out = jnp.concatenate(outs, axis=1)
    return jnp.transpose(out, (0, 2, 1))


# speedup vs baseline: 1.3895x; 1.3895x over previous
import functools

import jax
import jax.numpy as jnp
from jax import lax
from jax.experimental import pallas as pl
from jax.experimental.pallas import tpu as pltpu
from jax.experimental.pallas import tpu_sc as plsc

B, N1, N2, D, NS = 2, 2048, 2048, 64, 16
C1, C2 = 128, 128
BLK = 256
BLK2 = 128
NH = 4
N1H = N1 // NH
TOTH = B * N1H * NS
NW = 32
PER_W = TOTH // NW
CH = 128
NCH = PER_W // CH

_HI = jax.lax.Precision.HIGHEST


def _z_kernel(p2_ref, x2_ref, w1b_ref, w1c_ref, z_ref):
    x2 = x2_ref[0]
    w1c = w1c_ref[...]
    zc = (x2[:, 0:1] * w1c[0:1, :] + x2[:, 1:2] * w1c[1:2, :]
          + x2[:, 2:3] * w1c[2:3, :])
    z_ref[0] = jnp.dot(p2_ref[0], w1b_ref[...],
                       preferred_element_type=jnp.float32, precision=_HI) + zc


def _topk_kernel(x1_ref, x2_ref, idx_ref, w_ref):
    b = pl.program_id(0)
    x1 = x1_ref[0]
    x2 = x2_ref[0]
    x1sq = jnp.sum(x1 * x1, axis=1, keepdims=True)
    x2sq = jnp.sum(x2 * x2, axis=0, keepdims=True)
    x1b = x1.astype(jnp.bfloat16).astype(jnp.float32)
    x2b = x2.astype(jnp.bfloat16).astype(jnp.float32)
    cross_sel = (x1b[:, 0:1] * x2b[0:1, :] + x1b[:, 1:2] * x2b[1:2, :]
                 + x1b[:, 2:3] * x2b[2:3, :])
    sqd = (x1sq + x2sq) - 2.0 * cross_sel
    cross_ex = (x1[:, 0:1] * x2[0:1, :] + x1[:, 1:2] * x2[1:2, :]
                + x1[:, 2:3] * x2[2:3, :])
    sqde = (x1sq + x2sq) - 2.0 * cross_ex

    lane = jax.lax.broadcasted_iota(jnp.int32, (BLK, N2), 1)
    si = jax.lax.bitcast_convert_type(sqd, jnp.int32)
    key = si ^ ((si >> 31) & jnp.int32(0x7FFFFFFF))
    key = (key & jnp.int32(~0x7FF)) | lane
    idx_cols = []
    w_cols = []
    for _ in range(NS):
        km = jnp.min(key, axis=1, keepdims=True)
        hit = key == km
        idx = km & 0x7FF
        key = jnp.where(hit, jnp.int32(0x7FFFFFFF), key)
        me = jnp.sum(jnp.where(hit, sqde, 0.0), axis=1, keepdims=True)
        w = 1.0 / jnp.maximum(jnp.sqrt(jnp.maximum(me, 0.0)), 1e-10)
        idx_cols.append(idx + b * N2)
        w_cols.append(w)
    ws = jnp.concatenate(w_cols, axis=1)
    idx_ref[0] = jnp.concatenate(idx_cols, axis=1)
    w_ref[0] = ws / jnp.sum(ws, axis=1, keepdims=True)


def _mlp_kernel(gz_ref, x1_ref, p1_ref, w1a_ref, w1c_ref, b1_ref,
                w2_ref, b2_ref, wn_ref, out_ref):
    x1 = x1_ref[0]
    p1 = p1_ref[0]
    w1c = w1c_ref[...]
    basec = (x1[:, 0:1] * w1c[0:1, :] + x1[:, 1:2] * w1c[1:2, :]
             + x1[:, 2:3] * w1c[2:3, :])
    base = (jnp.dot(p1, w1a_ref[...], preferred_element_type=jnp.float32,
                    precision=_HI) - basec + b1_ref[...])
    base_e = jnp.broadcast_to(base[:, None, :], (BLK2, NS, C1)).reshape(
        BLK2 * NS, C1)
    h1 = gz_ref[0] + base_e
    h1 = jnp.where(h1 >= 0, h1, 0.1 * h1)
    h2 = jnp.dot(h1, w2_ref[...], preferred_element_type=jnp.float32,
                 precision=_HI) + b2_ref[...]
    h2 = jnp.where(h2 >= 0, h2, 0.1 * h2)
    h2 = h2.reshape(BLK2, NS, C2)
    out_ref[0] = jnp.sum(wn_ref[0][:, :, None] * h2, axis=1)


def _sc_gather(table, idx3):
    mesh = plsc.VectorSubcoreMesh(core_axis_name="c", subcore_axis_name="s")

    @functools.partial(
        pl.kernel, mesh=mesh,
        out_type=jax.ShapeDtypeStruct((TOTH, C1), jnp.float32),
        scratch_types=[
            pltpu.VMEM((NCH, CH), jnp.int32),
            pltpu.VMEM((CH, C1), jnp.float32),
            pltpu.VMEM((CH, C1), jnp.float32),
            pltpu.SemaphoreType.DMA,
            pltpu.SemaphoreType.DMA,
        ],
    )
    def k(table_hbm, idx_hbm, out_hbm, idx_v, rows_a, rows_b, sem_a, sem_b):
        wid = lax.axis_index("s") * 2 + lax.axis_index("c")
        base = wid * PER_W
        pltpu.sync_copy(idx_hbm.at[wid], idx_v)
        bufs = (rows_a, rows_b)
        sems = (sem_a, sem_b)
        cps = [None, None]
        for c in range(NCH + 1):
            if c < NCH:
                cps[c % 2] = pltpu.async_copy(table_hbm.at[idx_v.at[c]],
                                              bufs[c % 2], sems[c % 2])
            if c >= 1:
                cps[(c - 1) % 2].wait()
                pltpu.sync_copy(bufs[(c - 1) % 2],
                                out_hbm.at[pl.ds(base + (c - 1) * CH, CH)])

    return k(table, idx3)


def _half(z2d, xyz2, x1t_h, p1t_h, w1a, w1c, b1r, w2t, b2r):
    knn_idx, wn = pl.pallas_call(
        _topk_kernel,
        grid=(B, N1H // BLK),
        in_specs=[
            pl.BlockSpec((1, BLK, 3), lambda b, i: (b, i, 0)),
            pl.BlockSpec((1, 3, N2), lambda b, i: (b, 0, 0)),
        ],
        out_specs=(
            pl.BlockSpec((1, BLK, NS), lambda b, i: (b, i, 0)),
            pl.BlockSpec((1, BLK, NS), lambda b, i: (b, i, 0)),
        ),
        out_shape=(
            jax.ShapeDtypeStruct((B, N1H, NS), jnp.int32),
            jax.ShapeDtypeStruct((B, N1H, NS), jnp.float32),
        ),
    )(x1t_h, xyz2)

    gz = _sc_gather(z2d, knn_idx.reshape(NW, NCH, CH))

    return pl.pallas_call(
        _mlp_kernel,
        grid=(B, N1H // BLK2),
        in_specs=[
            pl.BlockSpec((1, BLK2 * NS, C1), lambda b, i: (b, i, 0)),
            pl.BlockSpec((1, BLK2, 3), lambda b, i: (b, i, 0)),
            pl.BlockSpec((1, BLK2, D), lambda b, i: (b, i, 0)),
            pl.BlockSpec((D, C1), lambda b, i: (0, 0)),
            pl.BlockSpec((3, C1), lambda b, i: (0, 0)),
            pl.BlockSpec((1, C1), lambda b, i: (0, 0)),
            pl.BlockSpec((C1, C2), lambda b, i: (0, 0)),
            pl.BlockSpec((1, C2), lambda b, i: (0, 0)),
            pl.BlockSpec((1, BLK2, NS), lambda b, i: (b, i, 0)),
        ],
        out_specs=pl.BlockSpec((1, BLK2, C2), lambda b, i: (b, i, 0)),
        out_shape=jax.ShapeDtypeStruct((B, N1H, C2), jnp.float32),
    )(gz.reshape(B, N1H * NS, C1), x1t_h, p1t_h, w1a, w1c, b1r, w2t, b2r, wn)


@jax.jit
def kernel(xyz1, xyz2, points1, points2, W1, b1, W2, b2):
    x1t = jnp.transpose(xyz1, (0, 2, 1))
    x2t = jnp.transpose(xyz2, (0, 2, 1))
    p1t = jnp.transpose(points1, (0, 2, 1))
    p2t = jnp.transpose(points2, (0, 2, 1))
    w1a = W1[:, :D].T
    w1b = W1[:, D:2 * D].T
    w1c = W1[:, 2 * D:].T
    w2t = W2.T
    b1r = b1.reshape(1, C1)
    b2r = b2.reshape(1, C2)

    z = pl.pallas_call(
        _z_kernel,
        grid=(B,),
        in_specs=[
            pl.BlockSpec((1, N2, D), lambda b: (b, 0, 0)),
            pl.BlockSpec((1, N2, 3), lambda b: (b, 0, 0)),
            pl.BlockSpec((D, C1), lambda b: (0, 0)),
            pl.BlockSpec((3, C1), lambda b: (0, 0)),
        ],
        out_specs=pl.BlockSpec((1, N2, C1), lambda b: (b, 0, 0)),
        out_shape=jax.ShapeDtypeStruct((B, N2, C1), jnp.float32),
    )(p2t, x2t, w1b, w1c)
    z2d = z.reshape(B * N2, C1)

    outs = [
        _half(z2d, xyz2, x1t[:, h * N1H:(h + 1) * N1H],
              p1t[:, h * N1H:(h + 1) * N1H], w1a, w1c, b1r, w2t, b2r)
        for h in range(NH)
    ]
    out = jnp.concatenate(outs, axis=1)
    return jnp.transpose(out, (0, 2, 1))


# speedup vs baseline: 1.4320x; 1.0306x over previous
import functools

import jax
import jax.numpy as jnp
from jax import lax
from jax.experimental import pallas as pl
from jax.experimental.pallas import tpu as pltpu
from jax.experimental.pallas import tpu_sc as plsc

B, N1, N2, D, NS = 2, 2048, 2048, 64, 16
C1, C2 = 128, 128
BLK = 256
BLK2 = 128
NH = 2
N1H = N1 // NH
TOTH = B * N1H * NS
NW = 32
PER_W = TOTH // NW
CH = 128
NCH = PER_W // CH

_HI = jax.lax.Precision.HIGHEST


def _z_kernel(p2_ref, x2_ref, w1b_ref, w1c_ref, z_ref):
    x2 = x2_ref[0]
    w1c = w1c_ref[...]
    zc = (x2[:, 0:1] * w1c[0:1, :] + x2[:, 1:2] * w1c[1:2, :]
          + x2[:, 2:3] * w1c[2:3, :])
    z_ref[0] = jnp.dot(p2_ref[0], w1b_ref[...],
                       preferred_element_type=jnp.float32, precision=_HI) + zc


def _topk_kernel(x1_ref, x2_ref, idx_ref, w_ref):
    b = pl.program_id(0)
    x1 = x1_ref[0]
    x2 = x2_ref[0]
    x1sq = jnp.sum(x1 * x1, axis=1, keepdims=True)
    x2sq = jnp.sum(x2 * x2, axis=0, keepdims=True)
    x1b = x1.astype(jnp.bfloat16).astype(jnp.float32)
    x2b = x2.astype(jnp.bfloat16).astype(jnp.float32)
    cross_sel = (x1b[:, 0:1] * x2b[0:1, :] + x1b[:, 1:2] * x2b[1:2, :]
                 + x1b[:, 2:3] * x2b[2:3, :])
    sqd = (x1sq + x2sq) - 2.0 * cross_sel
    cross_ex = (x1[:, 0:1] * x2[0:1, :] + x1[:, 1:2] * x2[1:2, :]
                + x1[:, 2:3] * x2[2:3, :])
    sqde = (x1sq + x2sq) - 2.0 * cross_ex

    lane = jax.lax.broadcasted_iota(jnp.int32, (BLK, N2), 1)
    si = jax.lax.bitcast_convert_type(sqd, jnp.int32)
    key = si ^ ((si >> 31) & jnp.int32(0x7FFFFFFF))
    key = (key & jnp.int32(~0x7FF)) | lane
    idx_cols = []
    w_cols = []
    for _ in range(NS):
        km = jnp.min(key, axis=1, keepdims=True)
        hit = key == km
        idx = km & 0x7FF
        key = jnp.where(hit, jnp.int32(0x7FFFFFFF), key)
        me = jnp.sum(jnp.where(hit, sqde, 0.0), axis=1, keepdims=True)
        w = 1.0 / jnp.maximum(jnp.sqrt(jnp.maximum(me, 0.0)), 1e-10)
        idx_cols.append(idx + b * N2)
        w_cols.append(w)
    ws = jnp.concatenate(w_cols, axis=1)
    idx_ref[0] = jnp.concatenate(idx_cols, axis=1)
    w_ref[0] = ws / jnp.sum(ws, axis=1, keepdims=True)


def _mlp_kernel(gz_ref, x1_ref, p1_ref, w1a_ref, w1c_ref, b1_ref,
                w2_ref, b2_ref, wn_ref, out_ref):
    x1 = x1_ref[0]
    p1 = p1_ref[0]
    w1c = w1c_ref[...]
    basec = (x1[:, 0:1] * w1c[0:1, :] + x1[:, 1:2] * w1c[1:2, :]
             + x1[:, 2:3] * w1c[2:3, :])
    base = (jnp.dot(p1, w1a_ref[...], preferred_element_type=jnp.float32,
                    precision=_HI) - basec + b1_ref[...])
    base_e = jnp.broadcast_to(base[:, None, :], (BLK2, NS, C1)).reshape(
        BLK2 * NS, C1)
    h1 = gz_ref[0] + base_e
    h1 = jnp.where(h1 >= 0, h1, 0.1 * h1)
    h2 = jnp.dot(h1, w2_ref[...], preferred_element_type=jnp.float32,
                 precision=_HI) + b2_ref[...]
    h2 = jnp.where(h2 >= 0, h2, 0.1 * h2)
    h2 = h2.reshape(BLK2, NS, C2)
    out_ref[0] = jnp.sum(wn_ref[0][:, :, None] * h2, axis=1)


def _sc_gather(table, idx3):
    mesh = plsc.VectorSubcoreMesh(core_axis_name="c", subcore_axis_name="s")

    @functools.partial(
        pl.kernel, mesh=mesh,
        out_type=jax.ShapeDtypeStruct((TOTH, C1), jnp.float32),
        scratch_types=[
            pltpu.VMEM((NCH, CH), jnp.int32),
            pltpu.VMEM((CH, C1), jnp.float32),
            pltpu.VMEM((CH, C1), jnp.float32),
            pltpu.SemaphoreType.DMA,
            pltpu.SemaphoreType.DMA,
        ],
    )
    def k(table_hbm, idx_hbm, out_hbm, idx_v, rows_a, rows_b, sem_a, sem_b):
        wid = lax.axis_index("s") * 2 + lax.axis_index("c")
        base = wid * PER_W
        pltpu.sync_copy(idx_hbm.at[wid], idx_v)
        bufs = (rows_a, rows_b)
        sems = (sem_a, sem_b)
        cps = [None, None]
        for c in range(NCH + 1):
            if c < NCH:
                cps[c % 2] = pltpu.async_copy(table_hbm.at[idx_v.at[c]],
                                              bufs[c % 2], sems[c % 2])
            if c >= 1:
                cps[(c - 1) % 2].wait()
                pltpu.sync_copy(bufs[(c - 1) % 2],
                                out_hbm.at[pl.ds(base + (c - 1) * CH, CH)])

    return k(table, idx3)


def _half(z2d, xyz2, x1t_h, p1t_h, w1a, w1c, b1r, w2t, b2r):
    knn_idx, wn = pl.pallas_call(
        _topk_kernel,
        grid=(B, N1H // BLK),
        in_specs=[
            pl.BlockSpec((1, BLK, 3), lambda b, i: (b, i, 0)),
            pl.BlockSpec((1, 3, N2), lambda b, i: (b, 0, 0)),
        ],
        out_specs=(
            pl.BlockSpec((1, BLK, NS), lambda b, i: (b, i, 0)),
            pl.BlockSpec((1, BLK, NS), lambda b, i: (b, i, 0)),
        ),
        out_shape=(
            jax.ShapeDtypeStruct((B, N1H, NS), jnp.int32),
            jax.ShapeDtypeStruct((B, N1H, NS), jnp.float32),
        ),
    )(x1t_h, xyz2)

    gz = _sc_gather(z2d, knn_idx.reshape(NW, NCH, CH))

    return pl.pallas_call(
        _mlp_kernel,
        grid=(B, N1H // BLK2),
        in_specs=[
            pl.BlockSpec((1, BLK2 * NS, C1), lambda b, i: (b, i, 0)),
            pl.BlockSpec((1, BLK2, 3), lambda b, i: (b, i, 0)),
            pl.BlockSpec((1, BLK2, D), lambda b, i: (b, i, 0)),
            pl.BlockSpec((D, C1), lambda b, i: (0, 0)),
            pl.BlockSpec((3, C1), lambda b, i: (0, 0)),
            pl.BlockSpec((1, C1), lambda b, i: (0, 0)),
            pl.BlockSpec((C1, C2), lambda b, i: (0, 0)),
            pl.BlockSpec((1, C2), lambda b, i: (0, 0)),
            pl.BlockSpec((1, BLK2, NS), lambda b, i: (b, i, 0)),
        ],
        out_specs=pl.BlockSpec((1, BLK2, C2), lambda b, i: (b, i, 0)),
        out_shape=jax.ShapeDtypeStruct((B, N1H, C2), jnp.float32),
    )(gz.reshape(B, N1H * NS, C1), x1t_h, p1t_h, w1a, w1c, b1r, w2t, b2r, wn)


@jax.jit
def kernel(xyz1, xyz2, points1, points2, W1, b1, W2, b2):
    x1t = jnp.transpose(xyz1, (0, 2, 1))
    x2t = jnp.transpose(xyz2, (0, 2, 1))
    p1t = jnp.transpose(points1, (0, 2, 1))
    p2t = jnp.transpose(points2, (0, 2, 1))
    w1a = W1[:, :D].T
    w1b = W1[:, D:2 * D].T
    w1c = W1[:, 2 * D:].T
    w2t = W2.T
    b1r = b1.reshape(1, C1)
    b2r = b2.reshape(1, C2)

    z = pl.pallas_call(
        _z_kernel,
        grid=(B,),
        in_specs=[
            pl.BlockSpec((1, N2, D), lambda b: (b, 0, 0)),
            pl.BlockSpec((1, N2, 3), lambda b: (b, 0, 0)),
            pl.BlockSpec((D, C1), lambda b: (0, 0)),
            pl.BlockSpec((3, C1), lambda b: (0, 0)),
        ],
        out_specs=pl.BlockSpec((1, N2, C1), lambda b: (b, 0, 0)),
        out_shape=jax.ShapeDtypeStruct((B, N2, C1), jnp.float32),
    )(p2t, x2t, w1b, w1c)
    z2d = z.reshape(B * N2, C1)

    outs = [
        _half(z2d, xyz2, x1t[:, h * N1H:(h + 1) * N1H],
              p1t[:, h * N1H:(h + 1) * N1H], w1a, w1c, b1r, w2t, b2r)
        for h in range(NH)
    ]
    out = jnp.concatenate(outs, axis=1)
    return jnp.transpose(out, (0, 2, 1))
